# baseline (speedup 1.0000x reference)
import functools

import jax
import jax.numpy as jnp
from jax import lax
from jax.experimental import pallas as pl
from jax.experimental.pallas import tpu as pltpu
from jax.experimental.pallas import tpu_sc as plsc

N = 10000
D = 256
CW = 128
N_PAD = 10240
E_TOT = 170000
BE = 128
NB_E = 84
EPT = NB_E * BE
E_PAD = 16 * EPT
RPT = N_PAD // 16
BN = 2560

_sc_mesh = dict(core_axis_name="c", subcore_axis_name="s")


@functools.cache
def _build_deg_kernel():
    return functools.partial(
        pl.kernel,
        mesh=plsc.VectorSubcoreMesh(**_sc_mesh),
        out_type=jax.ShapeDtypeStruct((2 * N_PAD, CW), jnp.float32),
        scratch_types=[
            pltpu.VMEM((4, NB_E // 4, BE), jnp.int32),
            pltpu.VMEM((BE, CW), jnp.float32),
            pltpu.VMEM_SHARED((N_PAD, CW), jnp.float32),
        ],
    )(_deg_body)


def _deg_body(dst_hbm, deg_out, idx_v, buf_v, accum):
    s = lax.axis_index("s")
    c = lax.axis_index("c")
    base = s * RPT

    def zrow(i, _):
        for j in range(CW // 16):
            buf_v[i, pl.ds(j * 16, 16)] = jnp.zeros((16,), jnp.float32)
        return 0

    lax.fori_loop(0, BE, zrow, 0)

    def zcp(i, _):
        pltpu.sync_copy(buf_v, accum.at[pl.ds(base + i * BE, BE)])
        return 0

    lax.fori_loop(0, RPT // BE, zcp, 0)
    pltpu.sync_copy(dst_hbm.at[s], idx_v)
    plsc.subcore_barrier()

    def orow(i, _):
        for j in range(CW // 16):
            buf_v[i, pl.ds(j * 16, 16)] = jnp.ones((16,), jnp.float32)
        return 0

    lax.fori_loop(0, BE, orow, 0)

    for p in range(2):
        def body(b, _):
            pltpu.sync_copy(buf_v, accum.at[idx_v.at[c * 2 + p, b]], add=True)
            return 0

        lax.fori_loop(0, NB_E // 4, body, 0)
    plsc.subcore_barrier()
    pltpu.sync_copy(
        accum.at[pl.ds(base, RPT)],
        deg_out.at[pl.ds(c * N_PAD + base, RPT)],
    )


@functools.cache
def _build_agg_kernel():
    return functools.partial(
        pl.kernel,
        mesh=plsc.VectorSubcoreMesh(**_sc_mesh),
        out_type=jax.ShapeDtypeStruct((2 * N_PAD, CW), jnp.float32),
        scratch_types=[
            pltpu.VMEM((NB_E // 4, BE), jnp.int32),
            pltpu.VMEM((NB_E // 4, BE), jnp.int32),
            pltpu.VMEM((2, BE, CW), jnp.float32),
            pltpu.VMEM_SHARED((N_PAD, CW), jnp.float32),
        ] + [pltpu.SemaphoreType.DMA] * 4,
    )(_agg_body)


def _agg_body(hs_hbm, src_hbm, dst_hbm, out_hbm, src_v, dst_v, bufs, accum,
              g0, g1, s0, s1):
    gs = (g0, g1)
    ss = (s0, s1)
    s = lax.axis_index("s")
    c = lax.axis_index("c")
    base = s * RPT

    def zrow(i, _):
        for j in range(CW // 16):
            bufs[0, i, pl.ds(j * 16, 16)] = jnp.zeros((16,), jnp.float32)
        return 0

    lax.fori_loop(0, BE, zrow, 0)

    def zcp(i, _):
        pltpu.sync_copy(bufs.at[0], accum.at[pl.ds(base + i * BE, BE)])
        return 0

    lax.fori_loop(0, RPT // BE, zcp, 0)
    plsc.subcore_barrier()

    def g_issue(b, k):
        pltpu.async_copy(hs_hbm.at[src_v.at[b]], bufs.at[k], gs[k])

    def g_wait(b, k):
        pltpu.make_async_copy(hs_hbm.at[src_v.at[b]], bufs.at[k], gs[k]).wait()

    def s_issue(b, k):
        pltpu.async_copy(bufs.at[k], accum.at[dst_v.at[b]], ss[k], add=True)

    def s_wait(b, k):
        pltpu.make_async_copy(bufs.at[k], accum.at[dst_v.at[b]], ss[k]).wait()

    NP4 = NB_E // 4
    for p in range(4):
        pltpu.sync_copy(src_hbm.at[c * 16 + s, p], src_v)
        pltpu.sync_copy(dst_hbm.at[s, p], dst_v)

        g_issue(0, 0)
        g_wait(0, 0); s_issue(0, 0); g_issue(1, 1)

        def body(r, _):
            for j in range(2):
                b = 2 * r + 1 + j
                k = (1 + j) % 2
                g_wait(b, k)
                s_issue(b, k)
                s_wait(b - 1, 1 - k)
                g_issue(b + 1, 1 - k)
            return 0

        nr = (NP4 - 3) // 2
        lax.fori_loop(0, nr, body, 0)

        for b in range(1 + 2 * nr, NP4):
            k = b % 2
            g_wait(b, k)
            s_issue(b, k)
            s_wait(b - 1, 1 - k)
            if b + 1 < NP4:
                g_issue(b + 1, 1 - k)
        s_wait(NP4 - 1, (NP4 - 1) % 2)
    plsc.subcore_barrier()
    pltpu.sync_copy(
        accum.at[pl.ds(base, RPT)],
        out_hbm.at[pl.ds(c * N_PAD + base, RPT)],
    )


def _dinv(d0_ref, d1_ref):
    deg = d0_ref[:, 0:1] + d1_ref[:, 0:1]
    return lax.rsqrt(jnp.maximum(deg, 1e-12))


def _prescale_body(x_ref, w_ref, d0_ref, d1_ref, o_ref):
    h = jnp.dot(x_ref[...], w_ref[...], preferred_element_type=jnp.float32)
    o_ref[...] = h * _dinv(d0_ref, d1_ref)


def _prescale(x_pad, W1, deg):
    return pl.pallas_call(
        _prescale_body,
        grid=(N_PAD // BN, 2),
        in_specs=[
            pl.BlockSpec((BN, D), lambda i, c: (i, 0)),
            pl.BlockSpec((D, CW), lambda i, c: (0, c)),
            pl.BlockSpec((BN, CW), lambda i, c: (i, 0)),
            pl.BlockSpec((BN, CW), lambda i, c: (N_PAD // BN + i, 0)),
        ],
        out_specs=pl.BlockSpec((BN, CW), lambda i, c: (c * (N_PAD // BN) + i, 0)),
        out_shape=jax.ShapeDtypeStruct((2 * N_PAD, CW), jnp.float32),
    )(x_pad, W1, deg, deg)


def _relu_cat(a0_ref, a1_ref, dinv, b_ref):
    h0 = jnp.maximum(a0_ref[...] * dinv + b_ref[0:1, :], 0.0)
    h1 = jnp.maximum(a1_ref[...] * dinv + b_ref[1:2, :], 0.0)
    return jnp.concatenate([h0, h1], axis=1)


def _mid_body(a0_ref, a1_ref, d0_ref, d1_ref, b_ref, w_ref, o_ref):
    dinv = _dinv(d0_ref, d1_ref)
    h = _relu_cat(a0_ref, a1_ref, dinv, b_ref)
    o_ref[...] = jnp.dot(h, w_ref[...], preferred_element_type=jnp.float32) * dinv


def _mid(agg, deg, b1r, W2):
    return pl.pallas_call(
        _mid_body,
        grid=(N_PAD // BN, 2),
        in_specs=[
            pl.BlockSpec((BN, CW), lambda i, c: (i, 0)),
            pl.BlockSpec((BN, CW), lambda i, c: (N_PAD // BN + i, 0)),
            pl.BlockSpec((BN, CW), lambda i, c: (i, 0)),
            pl.BlockSpec((BN, CW), lambda i, c: (N_PAD // BN + i, 0)),
            pl.BlockSpec((2, CW), lambda i, c: (0, 0)),
            pl.BlockSpec((D, CW), lambda i, c: (0, c)),
        ],
        out_specs=pl.BlockSpec((BN, CW), lambda i, c: (c * (N_PAD // BN) + i, 0)),
        out_shape=jax.ShapeDtypeStruct((2 * N_PAD, CW), jnp.float32),
    )(agg, agg, deg, deg, b1r, W2)


def _final_body(a0_ref, a1_ref, d0_ref, d1_ref, b_ref, wp_ref, bp_ref, o_ref):
    dinv = _dinv(d0_ref, d1_ref)
    h = _relu_cat(a0_ref, a1_ref, dinv, b_ref)
    z = jnp.dot(h, wp_ref[...], preferred_element_type=jnp.float32) + bp_ref[...]
    o_ref[...] = 1.0 / (1.0 + jnp.exp(-z))


def _final(agg, deg, b2r, wp_pad, bp_pad):
    return pl.pallas_call(
        _final_body,
        grid=(N_PAD // BN,),
        in_specs=[
            pl.BlockSpec((BN, CW), lambda i: (i, 0)),
            pl.BlockSpec((BN, CW), lambda i: (N_PAD // BN + i, 0)),
            pl.BlockSpec((BN, CW), lambda i: (i, 0)),
            pl.BlockSpec((BN, CW), lambda i: (N_PAD // BN + i, 0)),
            pl.BlockSpec((2, CW), lambda i: (0, 0)),
            pl.BlockSpec((D, CW), lambda i: (0, 0)),
            pl.BlockSpec((1, CW), lambda i: (0, 0)),
        ],
        out_specs=pl.BlockSpec((BN, CW), lambda i: (i, 0)),
        out_shape=jax.ShapeDtypeStruct((N_PAD, CW), jnp.float32),
    )(agg, agg, deg, deg, b2r, wp_pad, bp_pad)


def kernel(x, edge_index, W1, b1, W2, b2, Wp, bp):
    loop = jnp.arange(N, dtype=edge_index.dtype)
    pad = jnp.full((E_PAD - E_TOT,), N, dtype=edge_index.dtype)
    src = jnp.concatenate([edge_index[0], loop, pad])
    dst = jnp.concatenate([edge_index[1], loop, pad])
    src_r = src.reshape(16, 4, NB_E // 4, BE)
    dst_r = dst.reshape(16, 4, NB_E // 4, BE)
    src2 = jnp.concatenate([src_r, src_r + N_PAD], axis=0)

    x_pad = jnp.pad(x, ((0, N_PAD - N), (0, 0)))
    b1r = b1.reshape(2, CW)
    b2r = b2.reshape(2, CW)
    wp_pad = jnp.pad(Wp, ((0, 0), (0, CW - 1)))
    bp_pad = jnp.pad(bp, (0, CW - 1)).reshape(1, CW)

    deg = _deg_kernel(dst_r)
    hs1 = _prescale(x_pad, W1, deg)
    agg1 = _agg_kernel(hs1, src2, dst_r)
    hs2 = _mid(agg1, deg, b1r, W2)
    agg2 = _agg_kernel(hs2, src2, dst_r)
    out = _final(agg2, deg, b2r, wp_pad, bp_pad)
    return out[:N, 0:1]


def _deg_kernel(dst_r):
    return _build_deg_kernel()(dst_r)


def _agg_kernel(hs, src2, dst_r):
    return _build_agg_kernel()(hs, src2, dst_r)


# speedup vs baseline: 1.0169x; 1.0169x over previous
import functools

import jax
import jax.numpy as jnp
from jax import lax
from jax.experimental import pallas as pl
from jax.experimental.pallas import tpu as pltpu
from jax.experimental.pallas import tpu_sc as plsc

N = 10000
D = 256
CW = 128
N_PAD = 10240
E_TOT = 170000
BE = 64
NB_E = 168
EPT = NB_E * BE
E_PAD = 16 * EPT
RPT = N_PAD // 16
BN = 2560

_sc_mesh = dict(core_axis_name="c", subcore_axis_name="s")


@functools.cache
def _build_deg_kernel():
    return functools.partial(
        pl.kernel,
        mesh=plsc.VectorSubcoreMesh(**_sc_mesh),
        out_type=jax.ShapeDtypeStruct((2 * N_PAD, CW), jnp.float32),
        scratch_types=[
            pltpu.VMEM((4, NB_E // 4, BE), jnp.int32),
            pltpu.VMEM((BE, CW), jnp.float32),
            pltpu.VMEM_SHARED((N_PAD, CW), jnp.float32),
        ],
    )(_deg_body)


def _deg_body(dst_hbm, deg_out, idx_v, buf_v, accum):
    s = lax.axis_index("s")
    c = lax.axis_index("c")
    base = s * RPT

    def zrow(i, _):
        for j in range(CW // 16):
            buf_v[i, pl.ds(j * 16, 16)] = jnp.zeros((16,), jnp.float32)
        return 0

    lax.fori_loop(0, BE, zrow, 0)

    def zcp(i, _):
        pltpu.sync_copy(buf_v, accum.at[pl.ds(base + i * BE, BE)])
        return 0

    lax.fori_loop(0, RPT // BE, zcp, 0)
    pltpu.sync_copy(dst_hbm.at[s], idx_v)
    plsc.subcore_barrier()

    def orow(i, _):
        for j in range(CW // 16):
            buf_v[i, pl.ds(j * 16, 16)] = jnp.ones((16,), jnp.float32)
        return 0

    lax.fori_loop(0, BE, orow, 0)

    for p in range(2):
        def body(b, _):
            pltpu.sync_copy(buf_v, accum.at[idx_v.at[c * 2 + p, b]], add=True)
            return 0

        lax.fori_loop(0, NB_E // 4, body, 0)
    plsc.subcore_barrier()
    pltpu.sync_copy(
        accum.at[pl.ds(base, RPT)],
        deg_out.at[pl.ds(c * N_PAD + base, RPT)],
    )


@functools.cache
def _build_agg_kernel():
    return functools.partial(
        pl.kernel,
        mesh=plsc.VectorSubcoreMesh(**_sc_mesh),
        out_type=jax.ShapeDtypeStruct((2 * N_PAD, CW), jnp.float32),
        scratch_types=[
            pltpu.VMEM((NB_E // 4, BE), jnp.int32),
            pltpu.VMEM((NB_E // 4, BE), jnp.int32),
            pltpu.VMEM((4, BE, CW), jnp.float32),
            pltpu.VMEM_SHARED((N_PAD, CW), jnp.float32),
        ] + [pltpu.SemaphoreType.DMA] * 8,
    )(_agg_body)


def _agg_body(hs_hbm, src_hbm, dst_hbm, out_hbm, src_v, dst_v, bufs, accum,
              g0, g1, g2, g3, s0, s1, s2, s3):
    gs = (g0, g1, g2, g3)
    ss = (s0, s1, s2, s3)
    s = lax.axis_index("s")
    c = lax.axis_index("c")
    base = s * RPT

    def zrow(i, _):
        for j in range(CW // 16):
            bufs[0, i, pl.ds(j * 16, 16)] = jnp.zeros((16,), jnp.float32)
        return 0

    lax.fori_loop(0, BE, zrow, 0)

    def zcp(i, _):
        pltpu.sync_copy(bufs.at[0], accum.at[pl.ds(base + i * BE, BE)])
        return 0

    lax.fori_loop(0, RPT // BE, zcp, 0)
    plsc.subcore_barrier()

    def g_issue(b, k):
        pltpu.async_copy(hs_hbm.at[src_v.at[b]], bufs.at[k], gs[k])

    def g_wait(b, k):
        pltpu.make_async_copy(hs_hbm.at[src_v.at[b]], bufs.at[k], gs[k]).wait()

    def s_issue(b, k):
        pltpu.async_copy(bufs.at[k], accum.at[dst_v.at[b]], ss[k], add=True)

    def s_wait(b, k):
        pltpu.make_async_copy(bufs.at[k], accum.at[dst_v.at[b]], ss[k]).wait()

    NP4 = NB_E // 4
    for p in range(4):
        pltpu.sync_copy(src_hbm.at[c * 16 + s, p], src_v)
        pltpu.sync_copy(dst_hbm.at[s, p], dst_v)

        g_issue(0, 0)
        g_issue(1, 1)
        g_wait(0, 0); s_issue(0, 0); g_issue(2, 2)
        g_wait(1, 1); s_issue(1, 1); g_issue(3, 3)

        def body(r, _):
            for j in range(4):
                b = 4 * r + 2 + j
                k = (2 + j) % 4
                k2 = (k + 2) % 4
                g_wait(b, k)
                s_issue(b, k)
                s_wait(b - 2, k2)
                g_issue(b + 2, k2)
            return 0

        tail = 4 + (NP4 - 2) % 4
        nr = (NP4 - 2 - tail) // 4
        lax.fori_loop(0, nr, body, 0)

        for b in range(2 + 4 * nr, NP4):
            k = b % 4
            g_wait(b, k)
            s_issue(b, k)
            s_wait(b - 2, (k + 2) % 4)
            if b + 2 < NP4:
                g_issue(b + 2, (k + 2) % 4)
        s_wait(NP4 - 2, (NP4 - 2) % 4)
        s_wait(NP4 - 1, (NP4 - 1) % 4)
    plsc.subcore_barrier()
    pltpu.sync_copy(
        accum.at[pl.ds(base, RPT)],
        out_hbm.at[pl.ds(c * N_PAD + base, RPT)],
    )


def _dinv(d0_ref, d1_ref):
    deg = d0_ref[:, 0:1] + d1_ref[:, 0:1]
    return lax.rsqrt(jnp.maximum(deg, 1e-12))


def _prescale_body(x_ref, w_ref, d0_ref, d1_ref, o_ref):
    h = jnp.dot(x_ref[...], w_ref[...], preferred_element_type=jnp.float32)
    o_ref[...] = h * _dinv(d0_ref, d1_ref)


def _prescale(x_pad, W1, deg):
    return pl.pallas_call(
        _prescale_body,
        grid=(N_PAD // BN, 2),
        in_specs=[
            pl.BlockSpec((BN, D), lambda i, c: (i, 0)),
            pl.BlockSpec((D, CW), lambda i, c: (0, c)),
            pl.BlockSpec((BN, CW), lambda i, c: (i, 0)),
            pl.BlockSpec((BN, CW), lambda i, c: (N_PAD // BN + i, 0)),
        ],
        out_specs=pl.BlockSpec((BN, CW), lambda i, c: (c * (N_PAD // BN) + i, 0)),
        out_shape=jax.ShapeDtypeStruct((2 * N_PAD, CW), jnp.float32),
    )(x_pad, W1, deg, deg)


def _relu_cat(a0_ref, a1_ref, dinv, b_ref):
    h0 = jnp.maximum(a0_ref[...] * dinv + b_ref[0:1, :], 0.0)
    h1 = jnp.maximum(a1_ref[...] * dinv + b_ref[1:2, :], 0.0)
    return jnp.concatenate([h0, h1], axis=1)


def _mid_body(a0_ref, a1_ref, d0_ref, d1_ref, b_ref, w_ref, o_ref):
    dinv = _dinv(d0_ref, d1_ref)
    h = _relu_cat(a0_ref, a1_ref, dinv, b_ref)
    o_ref[...] = jnp.dot(h, w_ref[...], preferred_element_type=jnp.float32) * dinv


def _mid(agg, deg, b1r, W2):
    return pl.pallas_call(
        _mid_body,
        grid=(N_PAD // BN, 2),
        in_specs=[
            pl.BlockSpec((BN, CW), lambda i, c: (i, 0)),
            pl.BlockSpec((BN, CW), lambda i, c: (N_PAD // BN + i, 0)),
            pl.BlockSpec((BN, CW), lambda i, c: (i, 0)),
            pl.BlockSpec((BN, CW), lambda i, c: (N_PAD // BN + i, 0)),
            pl.BlockSpec((2, CW), lambda i, c: (0, 0)),
            pl.BlockSpec((D, CW), lambda i, c: (0, c)),
        ],
        out_specs=pl.BlockSpec((BN, CW), lambda i, c: (c * (N_PAD // BN) + i, 0)),
        out_shape=jax.ShapeDtypeStruct((2 * N_PAD, CW), jnp.float32),
    )(agg, agg, deg, deg, b1r, W2)


def _final_body(a0_ref, a1_ref, d0_ref, d1_ref, b_ref, wp_ref, bp_ref, o_ref):
    dinv = _dinv(d0_ref, d1_ref)
    h = _relu_cat(a0_ref, a1_ref, dinv, b_ref)
    z = jnp.dot(h, wp_ref[...], preferred_element_type=jnp.float32) + bp_ref[...]
    o_ref[...] = 1.0 / (1.0 + jnp.exp(-z))


def _final(agg, deg, b2r, wp_pad, bp_pad):
    return pl.pallas_call(
        _final_body,
        grid=(N_PAD // BN,),
        in_specs=[
            pl.BlockSpec((BN, CW), lambda i: (i, 0)),
            pl.BlockSpec((BN, CW), lambda i: (N_PAD // BN + i, 0)),
            pl.BlockSpec((BN, CW), lambda i: (i, 0)),
            pl.BlockSpec((BN, CW), lambda i: (N_PAD // BN + i, 0)),
            pl.BlockSpec((2, CW), lambda i: (0, 0)),
            pl.BlockSpec((D, CW), lambda i: (0, 0)),
            pl.BlockSpec((1, CW), lambda i: (0, 0)),
        ],
        out_specs=pl.BlockSpec((BN, CW), lambda i: (i, 0)),
        out_shape=jax.ShapeDtypeStruct((N_PAD, CW), jnp.float32),
    )(agg, agg, deg, deg, b2r, wp_pad, bp_pad)


def kernel(x, edge_index, W1, b1, W2, b2, Wp, bp):
    loop = jnp.arange(N, dtype=edge_index.dtype)
    pad = jnp.full((E_PAD - E_TOT,), N, dtype=edge_index.dtype)
    src = jnp.concatenate([edge_index[0], loop, pad])
    dst = jnp.concatenate([edge_index[1], loop, pad])
    src_r = src.reshape(16, 4, NB_E // 4, BE)
    dst_r = dst.reshape(16, 4, NB_E // 4, BE)
    src2 = jnp.concatenate([src_r, src_r + N_PAD], axis=0)

    x_pad = jnp.pad(x, ((0, N_PAD - N), (0, 0)))
    b1r = b1.reshape(2, CW)
    b2r = b2.reshape(2, CW)
    wp_pad = jnp.pad(Wp, ((0, 0), (0, CW - 1)))
    bp_pad = jnp.pad(bp, (0, CW - 1)).reshape(1, CW)

    deg = _deg_kernel(dst_r)
    hs1 = _prescale(x_pad, W1, deg)
    agg1 = _agg_kernel(hs1, src2, dst_r)
    hs2 = _mid(agg1, deg, b1r, W2)
    agg2 = _agg_kernel(hs2, src2, dst_r)
    out = _final(agg2, deg, b2r, wp_pad, bp_pad)
    return out[:N, 0:1]


def _deg_kernel(dst_r):
    return _build_deg_kernel()(dst_r)


def _agg_kernel(hs, src2, dst_r):
    return _build_agg_kernel()(hs, src2, dst_r)
